# baseline (speedup 1.0000x reference)
import functools

import jax
import jax.numpy as jnp
import numpy as np
from jax import lax
from jax.experimental import pallas as pl
from jax.experimental.pallas import tpu as pltpu
from jax.experimental.pallas import tpu_sc as plsc

_N = 10000
_E = 320000
_S = 4
_NMAX = 8
_RCUT = 5.0

_NC, _NS = 2, 16
_EPT = _E // _NS
_BLK = 2000
_NBLK = _EPT // _BLK
_G = 128
_GP = 80
_NPAIR = 7
_NSWEEP = 4
_COLS = 16
_ACC_ROWS = 40064
_ZBLK = _ACC_ROWS // _NS
_ZROWS = 626

_MU = [float(v) for v in np.linspace(0.0, _RCUT, _NMAX, dtype=np.float32)]
_INV_SIG = float(_NMAX / _RCUT)
_COS_C = [0.9999999922903372, -0.49999991771909824, 0.041666524352662083,
          -0.001388797034631234, 2.4773422692321623e-05,
          -2.711335744902814e-07, 1.7369072460331968e-09]


def _cos_poly(t):
    acc = jnp.full(t.shape, _COS_C[-1], jnp.float32)
    for a in _COS_C[-2::-1]:
        acc = acc * t + jnp.float32(a)
    return acc


def _sh_all(x, y, z):
    xx, yy, zz = x * x, y * y, z * z
    xy, yz, xz = x * y, y * z, x * z
    f5z2 = 5.0 * zz
    return [
        jnp.full(x.shape, 0.28209479177387814, jnp.float32),
        0.4886025119029199 * y,
        0.4886025119029199 * z,
        0.4886025119029199 * x,
        1.0925484305920792 * xy,
        1.0925484305920792 * yz,
        0.31539156525252005 * (3.0 * zz - 1.0),
        1.0925484305920792 * xz,
        0.5462742152960396 * (xx - yy),
        0.5900435899266435 * y * (3.0 * xx - yy),
        2.890611442640554 * xy * z,
        0.4570457994644658 * y * (f5z2 - 1.0),
        0.3731763325901154 * z * (f5z2 - 3.0),
        0.4570457994644658 * x * (f5z2 - 1.0),
        1.445305721320277 * z * (xx - yy),
        0.5900435899266435 * x * (xx - 3.0 * yy),
    ]


def _body(pos_h, spec_h, ctr_h, nbr_h, out_h,
          posf, spec, ctrb, nbrb, stgA, stgB, stgP, ixA, ixB, ixP, zb,
          acc, semA, semB):
    c = lax.axis_index("c")
    s = lax.axis_index("s")
    t0 = s * _EPT

    pltpu.sync_copy(pos_h, posf)
    pltpu.sync_copy(spec_h, spec)

    lane = lax.iota(jnp.int32, 16)
    is_sc0 = c == 0
    zero16f = jnp.zeros((16,), jnp.float32)

    def zinit(i, carry):
        zb[i] = zero16f
        return carry
    lax.fori_loop(0, _ZROWS, zinit, 0)
    for q in range(_ZBLK // _ZROWS):
        pltpu.sync_copy(zb, acc.at[pl.ds(s * _ZBLK + q * _ZROWS, _ZROWS)])

    def do_sub(loc, sweep, stg, ixr, sub16):
        ci = ctrb[pl.ds(loc, 16)]
        ni = nbrb[pl.ds(loc, 16)]
        ci3 = ci + ci + ci
        ni3 = ni + ni + ni
        cx = plsc.load_gather(posf, [ci3])
        cy = plsc.load_gather(posf, [ci3 + 1])
        cz = plsc.load_gather(posf, [ci3 + 2])
        nx = plsc.load_gather(posf, [ni3])
        ny = plsc.load_gather(posf, [ni3 + 1])
        nz = plsc.load_gather(posf, [ni3 + 2])
        sp = plsc.load_gather(spec, [ni])

        dx, dy, dz = nx - cx, ny - cy, nz - cz
        r2 = dx * dx + dy * dy + dz * dz + 1e-12
        ii = jnp.int32(0x5F3759DF) - lax.shift_right_logical(
            plsc.bitcast(r2, jnp.int32), 1)
        rv = plsc.bitcast(ii, jnp.float32)
        for _u in range(3):
            rv = rv * (1.5 - 0.5 * r2 * rv * rv)
        r = r2 * rv
        ux, uy, uz = dx * rv, dy * rv, dz * rv

        ta = jnp.minimum(r, _RCUT) * jnp.float32(np.pi / _RCUT)
        fc = 0.5 * (_cos_poly(ta * ta) + 1.0)
        rbs = []
        for n in range(_NMAX):
            tt = (r - _MU[n]) * _INV_SIG
            rbs.append(jnp.exp(-0.5 * (tt * tt)) * fc)

        sh = _sh_all(ux, uy, uz)
        comps = []
        for j in range(2):
            sel = [jnp.where(is_sc0, sh[2 * q + j], sh[8 + 2 * q + j])
                   for q in range(4)]
            comps.append(
                jnp.where(sweep == 0, sel[0],
                          jnp.where(sweep == 1, sel[1],
                                    jnp.where(sweep == 2, sel[2], sel[3]))))

        ixr[pl.ds(sub16, 16)] = ci * _S + sp
        rows_st = lane + sub16
        for j in range(2):
            for n in range(_NMAX):
                colv = jnp.full((16,), j * _NMAX + n, jnp.int32)
                plsc.store_scatter(stg, [rows_st, colv], comps[j] * rbs[n])

    def do_group(gbase, sweep, stg, ixr, nsub):
        for sub in range(nsub):
            do_sub(gbase + sub * 16, sweep, stg, ixr, sub * 16)

    def sweep_body(sweep, scarry):
        plsc.subcore_barrier()

        def blk_body(b, carry, sweep=sweep):
            pltpu.sync_copy(ctr_h.at[pl.ds(t0 + b * _BLK, _BLK)], ctrb)
            pltpu.sync_copy(nbr_h.at[pl.ds(t0 + b * _BLK, _BLK)], nbrb)

            def pair_body(p, carry2, b=b, sweep=sweep):
                @pl.when(jnp.logical_or(p > 0, b > 0))
                def _():
                    pltpu.make_async_copy(stgA, acc.at[ixA], semA).wait()
                do_group(2 * p * _G, sweep, stgA, ixA, 8)
                pltpu.async_copy(stgA, acc.at[ixA], semA, add=True)

                @pl.when(jnp.logical_or(p > 0, b > 0))
                def _():
                    pltpu.make_async_copy(stgB, acc.at[ixB], semB).wait()
                do_group((2 * p + 1) * _G, sweep, stgB, ixB, 8)
                pltpu.async_copy(stgB, acc.at[ixB], semB, add=True)
                return carry2

            lax.fori_loop(0, _NPAIR, pair_body, 0)

            pltpu.make_async_copy(stgA, acc.at[ixA], semA).wait()
            do_group(14 * _G, sweep, stgA, ixA, 8)
            pltpu.async_copy(stgA, acc.at[ixA], semA, add=True)
            do_group(15 * _G, sweep, stgP, ixP, _GP // 16)
            pltpu.sync_copy(stgP, acc.at[ixP], add=True)
            return carry

        lax.fori_loop(0, _NBLK, blk_body, 0)
        pltpu.make_async_copy(stgA, acc.at[ixA], semA).wait()
        pltpu.make_async_copy(stgB, acc.at[ixB], semB).wait()
        plsc.subcore_barrier()
        pltpu.sync_copy(
            acc.at[pl.ds(s * _ZBLK, _ZBLK)],
            out_h.at[c, sweep, pl.ds(s * _ZBLK, _ZBLK)])
        for q in range(_ZBLK // _ZROWS):
            pltpu.sync_copy(
                zb, acc.at[pl.ds(s * _ZBLK + q * _ZROWS, _ZROWS)])
        return scarry

    lax.fori_loop(0, _NSWEEP, sweep_body, 0)


@functools.lru_cache(maxsize=1)
def _get_expand():
    mesh = plsc.VectorSubcoreMesh(core_axis_name="c", subcore_axis_name="s",
                                  num_cores=_NC, num_subcores=_NS)
    return pl.kernel(
        _body,
        out_type=jax.ShapeDtypeStruct((_NC, _NSWEEP, _ACC_ROWS, _COLS),
                                      jnp.float32),
        mesh=mesh,
        compiler_params=pltpu.CompilerParams(
            needs_layout_passes=False, use_tc_tiling_on_sc=False),
        scratch_types=[
            pltpu.VMEM((3 * _N,), jnp.float32),
            pltpu.VMEM((_N,), jnp.int32),
            pltpu.VMEM((_BLK,), jnp.int32),
            pltpu.VMEM((_BLK,), jnp.int32),
            pltpu.VMEM((_G, _COLS), jnp.float32),
            pltpu.VMEM((_G, _COLS), jnp.float32),
            pltpu.VMEM((_GP, _COLS), jnp.float32),
            pltpu.VMEM((_G,), jnp.int32),
            pltpu.VMEM((_G,), jnp.int32),
            pltpu.VMEM((_GP,), jnp.int32),
            pltpu.VMEM((_ZROWS, _COLS), jnp.float32),
            pltpu.VMEM_SHARED((_ACC_ROWS, _COLS), jnp.float32),
            pltpu.SemaphoreType.DMA,
            pltpu.SemaphoreType.DMA,
        ],
    )


@jax.jit
def kernel(positions, edge_index, species):
    out8 = _get_expand()(positions.reshape(-1), species,
                         edge_index[0], edge_index[1])
    full = (out8.reshape(8, _ACC_ROWS, 2, _NMAX)[:, :_N * _S]
            .transpose(1, 0, 2, 3).reshape(_N, _S, 16, _NMAX))
    outs = []
    off = 0
    for l in range(4):
        m = 2 * l + 1
        outs.append(full[:, :, off:off + m, :].transpose(0, 2, 1, 3)
                    .reshape(_N, -1))
        off += m
    return jnp.concatenate(outs, axis=1)


# speedup vs baseline: 1.2860x; 1.2860x over previous
import functools

import jax
import jax.numpy as jnp
import numpy as np
from jax import lax
from jax.experimental import pallas as pl
from jax.experimental.pallas import tpu as pltpu
from jax.experimental.pallas import tpu_sc as plsc

_N = 10000
_E = 320000
_S = 4
_NMAX = 8
_RCUT = 5.0

_NC, _NS = 2, 16
_EPT = _E // _NS
_BLK = 2000
_NBLK = _EPT // _BLK
_G = 128
_GP = 80
_NPAIR = 7
_NSWEEP = 4
_COLS = 16
_ACC_ROWS = 40064
_ZBLK = _ACC_ROWS // _NS
_ZROWS = 626

_MU = [float(v) for v in np.linspace(0.0, _RCUT, _NMAX, dtype=np.float32)]
_INV_SIG = float(_NMAX / _RCUT)
_COS_C = [0.9999999922903372, -0.49999991771909824, 0.041666524352662083,
          -0.001388797034631234, 2.4773422692321623e-05,
          -2.711335744902814e-07, 1.7369072460331968e-09]


def _cos_poly(t):
    acc = jnp.full(t.shape, _COS_C[-1], jnp.float32)
    for a in _COS_C[-2::-1]:
        acc = acc * t + jnp.float32(a)
    return acc


def _sh_all(x, y, z):
    xx, yy, zz = x * x, y * y, z * z
    xy, yz, xz = x * y, y * z, x * z
    f5z2 = 5.0 * zz
    return [
        jnp.full(x.shape, 0.28209479177387814, jnp.float32),
        0.4886025119029199 * y,
        0.4886025119029199 * z,
        0.4886025119029199 * x,
        1.0925484305920792 * xy,
        1.0925484305920792 * yz,
        0.31539156525252005 * (3.0 * zz - 1.0),
        1.0925484305920792 * xz,
        0.5462742152960396 * (xx - yy),
        0.5900435899266435 * y * (3.0 * xx - yy),
        2.890611442640554 * xy * z,
        0.4570457994644658 * y * (f5z2 - 1.0),
        0.3731763325901154 * z * (f5z2 - 3.0),
        0.4570457994644658 * x * (f5z2 - 1.0),
        1.445305721320277 * z * (xx - yy),
        0.5900435899266435 * x * (xx - 3.0 * yy),
    ]


def _body(pos_h, spec_h, ctr_h, nbr_h, out_h,
          posf, spec, ctrb, nbrb, stgA, stgB, stgP, ixA, ixB, ixP, zb,
          acc, semA, semB):
    c = lax.axis_index("c")
    s = lax.axis_index("s")
    t0 = s * _EPT

    pltpu.sync_copy(pos_h, posf)
    pltpu.sync_copy(spec_h, spec)

    lane = lax.iota(jnp.int32, 16)
    is_sc0 = c == 0
    zero16f = jnp.zeros((16,), jnp.float32)

    def zinit(i, carry):
        zb[i] = zero16f
        return carry
    lax.fori_loop(0, _ZROWS, zinit, 0)
    for q in range(_ZBLK // _ZROWS):
        pltpu.sync_copy(zb, acc.at[pl.ds(s * _ZBLK + q * _ZROWS, _ZROWS)])

    def do_sub(loc, sweep, stg, ixr, sub16):
        ci = ctrb[pl.ds(loc, 16)]
        ni = nbrb[pl.ds(loc, 16)]
        ci3 = ci + ci + ci
        ni3 = ni + ni + ni
        cx = plsc.load_gather(posf, [ci3])
        cy = plsc.load_gather(posf, [ci3 + 1])
        cz = plsc.load_gather(posf, [ci3 + 2])
        nx = plsc.load_gather(posf, [ni3])
        ny = plsc.load_gather(posf, [ni3 + 1])
        nz = plsc.load_gather(posf, [ni3 + 2])
        sp = plsc.load_gather(spec, [ni])

        dx, dy, dz = nx - cx, ny - cy, nz - cz
        r2 = dx * dx + dy * dy + dz * dz + 1e-12
        ii = jnp.int32(0x5F3759DF) - lax.shift_right_logical(
            plsc.bitcast(r2, jnp.int32), 1)
        rv = plsc.bitcast(ii, jnp.float32)
        for _u in range(3):
            rv = rv * (1.5 - 0.5 * r2 * rv * rv)
        r = r2 * rv
        ux, uy, uz = dx * rv, dy * rv, dz * rv

        ta = jnp.minimum(r, _RCUT) * jnp.float32(np.pi / _RCUT)
        fc = 0.5 * (_cos_poly(ta * ta) + 1.0)
        rbs = []
        for n in range(_NMAX):
            tt = (r - _MU[n]) * _INV_SIG
            rbs.append(jnp.exp(-0.5 * (tt * tt)) * fc)

        sh = _sh_all(ux, uy, uz)
        comps = [
            jnp.where(is_sc0, sh[2 * sweep + j], sh[8 + 2 * sweep + j])
            for j in range(2)
        ]

        ixr[pl.ds(sub16, 16)] = ci * _S + sp
        rows_st = lane + sub16
        for j in range(2):
            for n in range(_NMAX):
                colv = jnp.full((16,), j * _NMAX + n, jnp.int32)
                plsc.store_scatter(stg, [rows_st, colv], comps[j] * rbs[n])

    def do_group(gbase, sweep, stg, ixr, nsub):
        for sub in range(nsub):
            do_sub(gbase + sub * 16, sweep, stg, ixr, sub * 16)

    for sweep in range(_NSWEEP):
        plsc.subcore_barrier()

        def blk_body(b, carry, sweep=sweep):
            pltpu.sync_copy(ctr_h.at[pl.ds(t0 + b * _BLK, _BLK)], ctrb)
            pltpu.sync_copy(nbr_h.at[pl.ds(t0 + b * _BLK, _BLK)], nbrb)

            def group_body(g, carry2, sweep=sweep):
                do_group(g * _G, sweep, stgA, ixA, 8)
                pltpu.sync_copy(stgA, acc.at[ixA], add=True)
                return carry2

            lax.fori_loop(0, 15, group_body, 0)
            do_group(15 * _G, sweep, stgP, ixP, _GP // 16)
            pltpu.sync_copy(stgP, acc.at[ixP], add=True)
            return carry

        lax.fori_loop(0, _NBLK, blk_body, 0)
        plsc.subcore_barrier()
        pltpu.sync_copy(
            acc.at[pl.ds(s * _ZBLK, _ZBLK)],
            out_h.at[c, sweep, pl.ds(s * _ZBLK, _ZBLK)])
        if sweep + 1 < _NSWEEP:
            for q in range(_ZBLK // _ZROWS):
                pltpu.sync_copy(
                    zb, acc.at[pl.ds(s * _ZBLK + q * _ZROWS, _ZROWS)])


@functools.lru_cache(maxsize=1)
def _get_expand():
    mesh = plsc.VectorSubcoreMesh(core_axis_name="c", subcore_axis_name="s",
                                  num_cores=_NC, num_subcores=_NS)
    return pl.kernel(
        _body,
        out_type=jax.ShapeDtypeStruct((_NC, _NSWEEP, _ACC_ROWS, _COLS),
                                      jnp.float32),
        mesh=mesh,
        compiler_params=pltpu.CompilerParams(
            needs_layout_passes=False, use_tc_tiling_on_sc=False),
        scratch_types=[
            pltpu.VMEM((3 * _N,), jnp.float32),
            pltpu.VMEM((_N,), jnp.int32),
            pltpu.VMEM((_BLK,), jnp.int32),
            pltpu.VMEM((_BLK,), jnp.int32),
            pltpu.VMEM((_G, _COLS), jnp.float32),
            pltpu.VMEM((_G, _COLS), jnp.float32),
            pltpu.VMEM((_GP, _COLS), jnp.float32),
            pltpu.VMEM((_G,), jnp.int32),
            pltpu.VMEM((_G,), jnp.int32),
            pltpu.VMEM((_GP,), jnp.int32),
            pltpu.VMEM((_ZROWS, _COLS), jnp.float32),
            pltpu.VMEM_SHARED((_ACC_ROWS, _COLS), jnp.float32),
            pltpu.SemaphoreType.DMA,
            pltpu.SemaphoreType.DMA,
        ],
    )


@jax.jit
def kernel(positions, edge_index, species):
    out8 = _get_expand()(positions.reshape(-1), species,
                         edge_index[0], edge_index[1])
    full = (out8.reshape(8, _ACC_ROWS, 2, _NMAX)[:, :_N * _S]
            .transpose(1, 0, 2, 3).reshape(_N, _S, 16, _NMAX))
    outs = []
    off = 0
    for l in range(4):
        m = 2 * l + 1
        outs.append(full[:, :, off:off + m, :].transpose(0, 2, 1, 3)
                    .reshape(_N, -1))
        off += m
    return jnp.concatenate(outs, axis=1)


# speedup vs baseline: 2.2506x; 1.7501x over previous
import functools

import jax
import jax.numpy as jnp
import numpy as np
from jax import lax
from jax.experimental import pallas as pl
from jax.experimental.pallas import tpu as pltpu
from jax.experimental.pallas import tpu_sc as plsc

_N = 10000
_E = 320000
_S = 4
_NMAX = 8
_RCUT = 5.0

_NC, _NS = 2, 16
_EPT = _E // _NS
_BLK = 2000
_NBLK = _EPT // _BLK
_G = 128
_GP = 80
_NPAIR = 7
_NSWEEP = 4
_COLS = 16
_ACC_ROWS = 40064
_ZBLK = _ACC_ROWS // _NS
_ZROWS = 626

_MU = [float(v) for v in np.linspace(0.0, _RCUT, _NMAX, dtype=np.float32)]
_INV_SIG = float(_NMAX / _RCUT)
_COS_C = [0.9999999922903372, -0.49999991771909824, 0.041666524352662083,
          -0.001388797034631234, 2.4773422692321623e-05,
          -2.711335744902814e-07, 1.7369072460331968e-09]


def _cos_poly(t):
    acc = jnp.full(t.shape, _COS_C[-1], jnp.float32)
    for a in _COS_C[-2::-1]:
        acc = acc * t + jnp.float32(a)
    return acc


def _sh_all(x, y, z):
    xx, yy, zz = x * x, y * y, z * z
    xy, yz, xz = x * y, y * z, x * z
    f5z2 = 5.0 * zz
    return [
        jnp.full(x.shape, 0.28209479177387814, jnp.float32),
        0.4886025119029199 * y,
        0.4886025119029199 * z,
        0.4886025119029199 * x,
        1.0925484305920792 * xy,
        1.0925484305920792 * yz,
        0.31539156525252005 * (3.0 * zz - 1.0),
        1.0925484305920792 * xz,
        0.5462742152960396 * (xx - yy),
        0.5900435899266435 * y * (3.0 * xx - yy),
        2.890611442640554 * xy * z,
        0.4570457994644658 * y * (f5z2 - 1.0),
        0.3731763325901154 * z * (f5z2 - 3.0),
        0.4570457994644658 * x * (f5z2 - 1.0),
        1.445305721320277 * z * (xx - yy),
        0.5900435899266435 * x * (xx - 3.0 * yy),
    ]


def _body(pos_h, spec_h, ctr_h, nbr_h, out_h,
          posf, spec, ctrb, nbrb, stgA, stgB, stgP, ixA, ixB, ixP, zb,
          acc, semA, semB):
    c = lax.axis_index("c")
    s = lax.axis_index("s")
    t0 = s * _EPT

    pltpu.sync_copy(pos_h, posf)
    pltpu.sync_copy(spec_h, spec)

    lane = lax.iota(jnp.int32, 16)
    is_sc0 = c == 0
    zero16f = jnp.zeros((16,), jnp.float32)

    def zinit(i, carry):
        zb[i] = zero16f
        return carry
    lax.fori_loop(0, _ZROWS, zinit, 0)
    for q in range(_ZBLK // _ZROWS):
        pltpu.sync_copy(zb, acc.at[pl.ds(s * _ZBLK + q * _ZROWS, _ZROWS)])

    def do_sub(loc, sweep, stg, ixr, sub16):
        ci = ctrb[pl.ds(loc, 16)]
        ni = nbrb[pl.ds(loc, 16)]
        ci3 = ci + ci + ci
        ni3 = ni + ni + ni
        cx = plsc.load_gather(posf, [ci3])
        cy = plsc.load_gather(posf, [ci3 + 1])
        cz = plsc.load_gather(posf, [ci3 + 2])
        nx = plsc.load_gather(posf, [ni3])
        ny = plsc.load_gather(posf, [ni3 + 1])
        nz = plsc.load_gather(posf, [ni3 + 2])
        sp = plsc.load_gather(spec, [ni])

        dx, dy, dz = nx - cx, ny - cy, nz - cz
        r2 = dx * dx + dy * dy + dz * dz + 1e-12
        ii = jnp.int32(0x5F3759DF) - lax.shift_right_logical(
            plsc.bitcast(r2, jnp.int32), 1)
        rv = plsc.bitcast(ii, jnp.float32)
        for _u in range(3):
            rv = rv * (1.5 - 0.5 * r2 * rv * rv)
        r = r2 * rv
        ux, uy, uz = dx * rv, dy * rv, dz * rv

        ta = jnp.minimum(r, _RCUT) * jnp.float32(np.pi / _RCUT)
        fc = 0.5 * (_cos_poly(ta * ta) + 1.0)
        rbs = []
        for n in range(_NMAX):
            tt = (r - _MU[n]) * _INV_SIG
            rbs.append(jnp.exp(-0.5 * (tt * tt)) * fc)

        sh = _sh_all(ux, uy, uz)
        comps = [
            jnp.where(is_sc0, sh[2 * sweep + j], sh[8 + 2 * sweep + j])
            for j in range(2)
        ]

        ixr[pl.ds(sub16, 16)] = ci * _S + sp
        rows_st = lane + sub16
        for j in range(2):
            for n in range(_NMAX):
                colv = jnp.full((16,), j * _NMAX + n, jnp.int32)
                plsc.store_scatter(stg, [rows_st, colv], comps[j] * rbs[n])

    def do_group(gbase, sweep, stg, ixr, nsub):
        for sub in range(nsub):
            do_sub(gbase + sub * 16, sweep, stg, ixr, sub * 16)

    for sweep in range(_NSWEEP):
        plsc.subcore_barrier()

        def blk_body(b, carry, sweep=sweep):
            pltpu.sync_copy(ctr_h.at[pl.ds(t0 + b * _BLK, _BLK)], ctrb)
            pltpu.sync_copy(nbr_h.at[pl.ds(t0 + b * _BLK, _BLK)], nbrb)

            def group_body(g, carry2, sweep=sweep):
                do_group(g * _G, sweep, stgA, ixA, 8)
                pltpu.sync_copy(stgA, acc.at[ixA], add=True)
                return carry2

            lax.fori_loop(0, 15, group_body, 0)
            do_group(15 * _G, sweep, stgP, ixP, _GP // 16)
            pltpu.sync_copy(stgP, acc.at[ixP], add=True)
            return carry

        lax.fori_loop(0, _NBLK, blk_body, 0)
        plsc.subcore_barrier()
        pltpu.sync_copy(
            acc.at[pl.ds(s * _ZBLK, _ZBLK)],
            out_h.at[c, sweep, pl.ds(s * _ZBLK, _ZBLK)])
        if sweep + 1 < _NSWEEP:
            for q in range(_ZBLK // _ZROWS):
                pltpu.sync_copy(
                    zb, acc.at[pl.ds(s * _ZBLK + q * _ZROWS, _ZROWS)])


@functools.lru_cache(maxsize=1)
def _get_expand():
    mesh = plsc.VectorSubcoreMesh(core_axis_name="c", subcore_axis_name="s",
                                  num_cores=_NC, num_subcores=_NS)
    return pl.kernel(
        _body,
        out_type=jax.ShapeDtypeStruct((_NC, _NSWEEP, _ACC_ROWS, _COLS),
                                      jnp.float32),
        mesh=mesh,
        compiler_params=pltpu.CompilerParams(
            needs_layout_passes=False, use_tc_tiling_on_sc=False),
        scratch_types=[
            pltpu.VMEM((3 * _N,), jnp.float32),
            pltpu.VMEM((_N,), jnp.int32),
            pltpu.VMEM((_BLK,), jnp.int32),
            pltpu.VMEM((_BLK,), jnp.int32),
            pltpu.VMEM((_G, _COLS), jnp.float32),
            pltpu.VMEM((_G, _COLS), jnp.float32),
            pltpu.VMEM((_GP, _COLS), jnp.float32),
            pltpu.VMEM((_G,), jnp.int32),
            pltpu.VMEM((_G,), jnp.int32),
            pltpu.VMEM((_GP,), jnp.int32),
            pltpu.VMEM((_ZROWS, _COLS), jnp.float32),
            pltpu.VMEM_SHARED((_ACC_ROWS, _COLS), jnp.float32),
            pltpu.SemaphoreType.DMA,
            pltpu.SemaphoreType.DMA,
        ],
    )


_PRB = 2000


def _permute_body(in_ref, out_ref):
    halves = []
    for h in range(2):
        x = in_ref[h]
        pieces = [x[:, a * 16 + j * 8:a * 16 + j * 8 + 8]
                  for j in range(2) for a in range(_S)]
        halves.append(jnp.concatenate(pieces, axis=1))
    out_ref[...] = jnp.concatenate(halves, axis=1)


@functools.lru_cache(maxsize=1)
def _get_permute():
    return pl.pallas_call(
        _permute_body,
        grid=(4, _N // _PRB),
        in_specs=[pl.BlockSpec((2, _PRB, 64), lambda i, r: (i, r, 0))],
        out_specs=pl.BlockSpec((_PRB, 128), lambda i, r: (r, i)),
        out_shape=jax.ShapeDtypeStruct((_N, 512), jnp.float32),
    )


@jax.jit
def kernel(positions, edge_index, species):
    out8 = _get_expand()(positions.reshape(-1), species,
                         edge_index[0], edge_index[1])
    return _get_permute()(out8.reshape(8, _ACC_ROWS * _COLS // 64, 64))
